# Initial kernel scaffold; baseline (speedup 1.0000x reference)
import jax
import jax.numpy as jnp
from jax.experimental import pallas as pl


def kernel(input_ids, word_table, pretrained_table, pos_table, gamma, beta):
    raise NotImplementedError("write your pallas kernel here")


# speedup vs baseline: 7.8442x; 7.8442x over previous
import functools

import jax
import jax.numpy as jnp
from jax import lax
from jax.experimental import pallas as pl
from jax.experimental.pallas import tpu as pltpu
from jax.experimental.pallas import tpu_sc as plsc

_EPS = 1e-12


def _sc_gather(ids_flat, word_table):
    N = ids_flat.shape[0]
    H = word_table.shape[1]
    info = plsc.get_sparse_core_info()
    NC, NS = info.num_cores, info.num_subcores
    NW = NC * NS
    per_w = N // NW
    CH = 128
    n_ch = per_w // CH

    mesh = plsc.VectorSubcoreMesh(core_axis_name="c", subcore_axis_name="s")

    @functools.partial(
        pl.kernel,
        mesh=mesh,
        out_type=jax.ShapeDtypeStruct((N, H), jnp.float32),
        scratch_types=[
            pltpu.VMEM((CH,), jnp.int32),
            pltpu.VMEM((CH, H), jnp.float32),
            pltpu.SemaphoreType.DMA,
        ],
    )
    def k(ids_hbm, table_hbm, out_hbm, idx_v, rows_v, sem):
        wid = lax.axis_index("s") * NC + lax.axis_index("c")
        base = wid * per_w

        def body(g, carry):
            off = base + g * CH
            pltpu.sync_copy(ids_hbm.at[pl.ds(off, CH)], idx_v)
            pltpu.async_copy(table_hbm.at[idx_v], rows_v, sem).wait()
            pltpu.sync_copy(rows_v, out_hbm.at[pl.ds(off, CH)])
            return carry

        lax.fori_loop(0, n_ch, body, 0)

    return k(ids_flat, word_table)


def _tc_add_layernorm(gathered, pre_tab, pos_tab, gamma, beta):
    B, L, H = gathered.shape
    RB = 8
    grid = (B // RB,)

    def body(g_ref, pa_ref, pb_ref, gm_ref, bt_ref, o_ref):
        x = g_ref[...] + (pa_ref[...] + pb_ref[...])[None, :, :]
        u = jnp.mean(x, axis=-1, keepdims=True)
        s2 = jnp.mean((x - u) ** 2, axis=-1, keepdims=True)
        xn = (x - u) * lax.rsqrt(s2 + _EPS)
        o_ref[...] = xn * gm_ref[0][None, None, :] + bt_ref[0][None, None, :]

    return pl.pallas_call(
        body,
        grid=grid,
        in_specs=[
            pl.BlockSpec((RB, L, H), lambda i: (i, 0, 0)),
            pl.BlockSpec((L, H), lambda i: (0, 0)),
            pl.BlockSpec((L, H), lambda i: (0, 0)),
            pl.BlockSpec((1, H), lambda i: (0, 0)),
            pl.BlockSpec((1, H), lambda i: (0, 0)),
        ],
        out_specs=pl.BlockSpec((RB, L, H), lambda i: (i, 0, 0)),
        out_shape=jax.ShapeDtypeStruct((B, L, H), jnp.float32),
    )(gathered, pre_tab, pos_tab, gamma, beta)


def kernel(input_ids, word_table, pretrained_table, pos_table, gamma, beta):
    B, L = input_ids.shape
    H = word_table.shape[1]
    ids_flat = input_ids.reshape(-1).astype(jnp.int32)
    gathered = _sc_gather(ids_flat, word_table)
    return _tc_add_layernorm(
        gathered.reshape(B, L, H),
        pretrained_table[:L],
        pos_table[:L],
        gamma.reshape(1, H),
        beta.reshape(1, H),
    )


# speedup vs baseline: 9.8761x; 1.2590x over previous
import functools

import jax
import jax.numpy as jnp
from jax import lax
from jax.experimental import pallas as pl
from jax.experimental.pallas import tpu as pltpu
from jax.experimental.pallas import tpu_sc as plsc

_EPS = 1e-12


def _sc_gather(ids_flat, word_table):
    N = ids_flat.shape[0]
    H = word_table.shape[1]
    info = plsc.get_sparse_core_info()
    NC, NS = info.num_cores, info.num_subcores
    NW = NC * NS
    per_w = N // NW
    CH = 128
    n_ch = per_w // CH

    mesh = plsc.VectorSubcoreMesh(core_axis_name="c", subcore_axis_name="s")
    NB = 2

    @functools.partial(
        pl.kernel,
        mesh=mesh,
        out_type=jax.ShapeDtypeStruct((N, H), jnp.float32),
        scratch_types=[
            pltpu.VMEM((NB, CH), jnp.int32),
            pltpu.VMEM((NB, CH, H), jnp.float32),
            pltpu.SemaphoreType.DMA,
            pltpu.SemaphoreType.DMA,
            pltpu.SemaphoreType.DMA,
            pltpu.SemaphoreType.DMA,
        ],
    )
    def k(ids_hbm, table_hbm, out_hbm, idx_v, rows_v, g0, g1, w0, w1):
        wid = lax.axis_index("s") * NC + lax.axis_index("c")
        base = wid * per_w
        gsem = (g0, g1)
        wsem = (w0, w1)

        for b in range(NB):
            off = base + b * CH
            pltpu.sync_copy(ids_hbm.at[pl.ds(off, CH)], idx_v.at[b])
            pltpu.async_copy(table_hbm.at[idx_v.at[b]], rows_v.at[b], gsem[b])

        def body(i, carry):
            for b in range(NB):
                g = i * NB + b
                off = base + g * CH
                pltpu.make_async_copy(
                    table_hbm.at[idx_v.at[b]], rows_v.at[b], gsem[b]
                ).wait()
                pltpu.async_copy(
                    rows_v.at[b], out_hbm.at[pl.ds(off, CH)], wsem[b]
                )

                @pl.when(i < n_ch // NB - 1)
                def _prefetch():
                    noff = off + NB * CH
                    pltpu.sync_copy(ids_hbm.at[pl.ds(noff, CH)], idx_v.at[b])
                    pltpu.make_async_copy(
                        rows_v.at[b], out_hbm.at[pl.ds(off, CH)], wsem[b]
                    ).wait()
                    pltpu.async_copy(
                        table_hbm.at[idx_v.at[b]], rows_v.at[b], gsem[b]
                    )

            return carry

        lax.fori_loop(0, n_ch // NB, body, 0)
        for b in range(NB):
            off = base + (n_ch - NB + b) * CH
            pltpu.make_async_copy(
                rows_v.at[b], out_hbm.at[pl.ds(off, CH)], wsem[b]
            ).wait()

    return k(ids_flat, word_table)


def _tc_add_layernorm(gathered, pre_tab, pos_tab, gamma, beta):
    B, L, H = gathered.shape
    RB = 8
    grid = (B // RB,)

    def body(g_ref, pa_ref, pb_ref, gm_ref, bt_ref, o_ref):
        x = g_ref[...] + (pa_ref[...] + pb_ref[...])[None, :, :]
        u = jnp.mean(x, axis=-1, keepdims=True)
        s2 = jnp.mean((x - u) ** 2, axis=-1, keepdims=True)
        xn = (x - u) * lax.rsqrt(s2 + _EPS)
        o_ref[...] = xn * gm_ref[0][None, None, :] + bt_ref[0][None, None, :]

    return pl.pallas_call(
        body,
        grid=grid,
        in_specs=[
            pl.BlockSpec((RB, L, H), lambda i: (i, 0, 0)),
            pl.BlockSpec((L, H), lambda i: (0, 0)),
            pl.BlockSpec((L, H), lambda i: (0, 0)),
            pl.BlockSpec((1, H), lambda i: (0, 0)),
            pl.BlockSpec((1, H), lambda i: (0, 0)),
        ],
        out_specs=pl.BlockSpec((RB, L, H), lambda i: (i, 0, 0)),
        out_shape=jax.ShapeDtypeStruct((B, L, H), jnp.float32),
    )(gathered, pre_tab, pos_tab, gamma, beta)


def kernel(input_ids, word_table, pretrained_table, pos_table, gamma, beta):
    B, L = input_ids.shape
    H = word_table.shape[1]
    ids_flat = input_ids.reshape(-1).astype(jnp.int32)
    gathered = _sc_gather(ids_flat, word_table)
    return _tc_add_layernorm(
        gathered.reshape(B, L, H),
        pretrained_table[:L],
        pos_table[:L],
        gamma.reshape(1, H),
        beta.reshape(1, H),
    )


# speedup vs baseline: 10.0124x; 1.0138x over previous
import functools

import jax
import jax.numpy as jnp
from jax import lax
from jax.experimental import pallas as pl
from jax.experimental.pallas import tpu as pltpu
from jax.experimental.pallas import tpu_sc as plsc

_EPS = 1e-12


def _sc_gather(ids_flat, word_table):
    N = ids_flat.shape[0]
    H = word_table.shape[1]
    info = plsc.get_sparse_core_info()
    NC, NS = info.num_cores, info.num_subcores
    NW = NC * NS
    per_w = N // NW
    CH = 128
    n_ch = per_w // CH

    mesh = plsc.VectorSubcoreMesh(core_axis_name="c", subcore_axis_name="s")
    NB = 5

    @functools.partial(
        pl.kernel,
        mesh=mesh,
        out_type=jax.ShapeDtypeStruct((N, H), jnp.float32),
        scratch_types=[
            pltpu.VMEM((NB, CH), jnp.int32),
            pltpu.VMEM((NB, CH, H), jnp.float32),
        ]
        + [pltpu.SemaphoreType.DMA] * (2 * NB),
    )
    def k(ids_hbm, table_hbm, out_hbm, idx_v, rows_v, *sems):
        wid = lax.axis_index("s") * NC + lax.axis_index("c")
        base = wid * per_w
        gsem = sems[:NB]
        wsem = sems[NB:]

        for b in range(NB):
            off = base + b * CH
            pltpu.sync_copy(ids_hbm.at[pl.ds(off, CH)], idx_v.at[b])
            pltpu.async_copy(table_hbm.at[idx_v.at[b]], rows_v.at[b], gsem[b])

        def body(i, carry):
            for b in range(NB):
                g = i * NB + b
                off = base + g * CH
                pltpu.make_async_copy(
                    table_hbm.at[idx_v.at[b]], rows_v.at[b], gsem[b]
                ).wait()
                pltpu.async_copy(
                    rows_v.at[b], out_hbm.at[pl.ds(off, CH)], wsem[b]
                )

                @pl.when(i < n_ch // NB - 1)
                def _prefetch():
                    noff = off + NB * CH
                    pltpu.sync_copy(ids_hbm.at[pl.ds(noff, CH)], idx_v.at[b])
                    pltpu.make_async_copy(
                        rows_v.at[b], out_hbm.at[pl.ds(off, CH)], wsem[b]
                    ).wait()
                    pltpu.async_copy(
                        table_hbm.at[idx_v.at[b]], rows_v.at[b], gsem[b]
                    )

            return carry

        lax.fori_loop(0, n_ch // NB, body, 0)
        for b in range(NB):
            off = base + (n_ch - NB + b) * CH
            pltpu.make_async_copy(
                rows_v.at[b], out_hbm.at[pl.ds(off, CH)], wsem[b]
            ).wait()

    return k(ids_flat, word_table)


def _tc_add_layernorm(gathered, pre_tab, pos_tab, gamma, beta):
    B, L, H = gathered.shape
    RB = 8
    grid = (B // RB,)

    def body(g_ref, pa_ref, pb_ref, gm_ref, bt_ref, o_ref):
        x = g_ref[...] + (pa_ref[...] + pb_ref[...])[None, :, :]
        u = jnp.mean(x, axis=-1, keepdims=True)
        s2 = jnp.mean((x - u) ** 2, axis=-1, keepdims=True)
        xn = (x - u) * lax.rsqrt(s2 + _EPS)
        o_ref[...] = xn * gm_ref[0][None, None, :] + bt_ref[0][None, None, :]

    return pl.pallas_call(
        body,
        grid=grid,
        in_specs=[
            pl.BlockSpec((RB, L, H), lambda i: (i, 0, 0)),
            pl.BlockSpec((L, H), lambda i: (0, 0)),
            pl.BlockSpec((L, H), lambda i: (0, 0)),
            pl.BlockSpec((1, H), lambda i: (0, 0)),
            pl.BlockSpec((1, H), lambda i: (0, 0)),
        ],
        out_specs=pl.BlockSpec((RB, L, H), lambda i: (i, 0, 0)),
        out_shape=jax.ShapeDtypeStruct((B, L, H), jnp.float32),
    )(gathered, pre_tab, pos_tab, gamma, beta)


def kernel(input_ids, word_table, pretrained_table, pos_table, gamma, beta):
    B, L = input_ids.shape
    H = word_table.shape[1]
    ids_flat = input_ids.reshape(-1).astype(jnp.int32)
    gathered = _sc_gather(ids_flat, word_table)
    return _tc_add_layernorm(
        gathered.reshape(B, L, H),
        pretrained_table[:L],
        pos_table[:L],
        gamma.reshape(1, H),
        beta.reshape(1, H),
    )


# speedup vs baseline: 12.8949x; 1.2879x over previous
import functools

import jax
import jax.numpy as jnp
from jax import lax
from jax.experimental import pallas as pl
from jax.experimental.pallas import tpu as pltpu
from jax.experimental.pallas import tpu_sc as plsc

_EPS = 1e-12


def _sc_gather(ids_flat, word_table):
    N = ids_flat.shape[0]
    H = word_table.shape[1]
    info = plsc.get_sparse_core_info()
    NC, NS = info.num_cores, info.num_subcores
    NW = NC * NS
    per_w = N // NW
    CH = 128
    n_ch = per_w // CH

    mesh = plsc.VectorSubcoreMesh(core_axis_name="c", subcore_axis_name="s")
    NB = 5

    @functools.partial(
        pl.kernel,
        mesh=mesh,
        out_type=jax.ShapeDtypeStruct((N, H), jnp.float32),
        scratch_types=[
            pltpu.VMEM((NB, CH), jnp.int32),
            pltpu.VMEM((NB, CH, H), jnp.float32),
        ]
        + [pltpu.SemaphoreType.DMA] * (2 * NB),
    )
    def k(ids_hbm, table_hbm, out_hbm, idx_v, rows_v, *sems):
        wid = lax.axis_index("s") * NC + lax.axis_index("c")
        base = wid * per_w
        gsem = sems[:NB]
        wsem = sems[NB:]

        for b in range(NB):
            off = base + b * CH
            pltpu.sync_copy(ids_hbm.at[pl.ds(off, CH)], idx_v.at[b])
            pltpu.async_copy(table_hbm.at[idx_v.at[b]], rows_v.at[b], gsem[b])

        def body(i, carry):
            for b in range(NB):
                g = i * NB + b
                off = base + g * CH
                pltpu.make_async_copy(
                    table_hbm.at[idx_v.at[b]], rows_v.at[b], gsem[b]
                ).wait()
                pltpu.async_copy(
                    rows_v.at[b], out_hbm.at[pl.ds(off, CH)], wsem[b]
                )

                @pl.when(i < n_ch // NB - 1)
                def _prefetch():
                    noff = off + NB * CH
                    pltpu.sync_copy(ids_hbm.at[pl.ds(noff, CH)], idx_v.at[b])
                    pltpu.make_async_copy(
                        rows_v.at[b], out_hbm.at[pl.ds(off, CH)], wsem[b]
                    ).wait()
                    pltpu.async_copy(
                        table_hbm.at[idx_v.at[b]], rows_v.at[b], gsem[b]
                    )

            return carry

        lax.fori_loop(0, n_ch // NB, body, 0)
        for b in range(NB):
            off = base + (n_ch - NB + b) * CH
            pltpu.make_async_copy(
                rows_v.at[b], out_hbm.at[pl.ds(off, CH)], wsem[b]
            ).wait()

    return k(ids_flat, word_table)


def _tc_add_layernorm(gathered, pre_tab, pos_tab, gamma, beta):
    B, L, H = gathered.shape
    RB = 32
    grid = (B // RB,)

    def body(g_ref, pa_ref, pb_ref, gm_ref, bt_ref, o_ref):
        x = g_ref[...] + (pa_ref[...] + pb_ref[...])[None, :, :]
        u = jnp.mean(x, axis=-1, keepdims=True)
        s2 = jnp.mean((x - u) ** 2, axis=-1, keepdims=True)
        xn = (x - u) * lax.rsqrt(s2 + _EPS)
        o_ref[...] = xn * gm_ref[0][None, None, :] + bt_ref[0][None, None, :]

    return pl.pallas_call(
        body,
        grid=grid,
        in_specs=[
            pl.BlockSpec((RB, L, H), lambda i: (i, 0, 0)),
            pl.BlockSpec((L, H), lambda i: (0, 0)),
            pl.BlockSpec((L, H), lambda i: (0, 0)),
            pl.BlockSpec((1, H), lambda i: (0, 0)),
            pl.BlockSpec((1, H), lambda i: (0, 0)),
        ],
        out_specs=pl.BlockSpec((RB, L, H), lambda i: (i, 0, 0)),
        out_shape=jax.ShapeDtypeStruct((B, L, H), jnp.float32),
    )(gathered, pre_tab, pos_tab, gamma, beta)


def kernel(input_ids, word_table, pretrained_table, pos_table, gamma, beta):
    B, L = input_ids.shape
    H = word_table.shape[1]
    ids_flat = input_ids.reshape(-1).astype(jnp.int32)
    gathered = _sc_gather(ids_flat, word_table)
    return _tc_add_layernorm(
        gathered.reshape(B, L, H),
        pretrained_table[:L],
        pos_table[:L],
        gamma.reshape(1, H),
        beta.reshape(1, H),
    )


# speedup vs baseline: 13.5434x; 1.0503x over previous
import functools

import jax
import jax.numpy as jnp
from jax import lax
from jax.experimental import pallas as pl
from jax.experimental.pallas import tpu as pltpu
from jax.experimental.pallas import tpu_sc as plsc

_EPS = 1e-12


def _sc_gather(ids_flat, word_table):
    N = ids_flat.shape[0]
    H = word_table.shape[1]
    info = plsc.get_sparse_core_info()
    NC, NS = info.num_cores, info.num_subcores
    NW = NC * NS
    per_w = N // NW
    CH = 128
    n_ch = per_w // CH

    mesh = plsc.VectorSubcoreMesh(core_axis_name="c", subcore_axis_name="s")
    NB = 5

    @functools.partial(
        pl.kernel,
        mesh=mesh,
        out_type=jax.ShapeDtypeStruct((N, H), jnp.float32),
        scratch_types=[
            pltpu.VMEM((NB, CH), jnp.int32),
            pltpu.VMEM((NB, CH, H), jnp.float32),
        ]
        + [pltpu.SemaphoreType.DMA] * (2 * NB),
    )
    def k(ids_hbm, table_hbm, out_hbm, idx_v, rows_v, *sems):
        wid = lax.axis_index("s") * NC + lax.axis_index("c")
        base = wid * per_w
        gsem = sems[:NB]
        wsem = sems[NB:]

        for b in range(NB):
            off = base + b * CH
            pltpu.sync_copy(ids_hbm.at[pl.ds(off, CH)], idx_v.at[b])
            pltpu.async_copy(table_hbm.at[idx_v.at[b]], rows_v.at[b], gsem[b])

        def body(i, carry):
            for b in range(NB):
                g = i * NB + b
                off = base + g * CH
                pltpu.make_async_copy(
                    table_hbm.at[idx_v.at[b]], rows_v.at[b], gsem[b]
                ).wait()
                pltpu.async_copy(
                    rows_v.at[b], out_hbm.at[pl.ds(off, CH)], wsem[b]
                )

                @pl.when(i < n_ch // NB - 1)
                def _prefetch():
                    noff = off + NB * CH
                    pltpu.sync_copy(ids_hbm.at[pl.ds(noff, CH)], idx_v.at[b])
                    pltpu.make_async_copy(
                        rows_v.at[b], out_hbm.at[pl.ds(off, CH)], wsem[b]
                    ).wait()
                    pltpu.async_copy(
                        table_hbm.at[idx_v.at[b]], rows_v.at[b], gsem[b]
                    )

            return carry

        lax.fori_loop(0, n_ch // NB, body, 0)
        for b in range(NB):
            off = base + (n_ch - NB + b) * CH
            pltpu.make_async_copy(
                rows_v.at[b], out_hbm.at[pl.ds(off, CH)], wsem[b]
            ).wait()

    return k(ids_flat, word_table)


def _tc_add_layernorm(gathered, pre_tab, pos_tab, gamma, beta):
    B, L, H = gathered.shape
    RB = 64
    grid = (B // RB,)

    def body(g_ref, pa_ref, pb_ref, gm_ref, bt_ref, o_ref):
        x = g_ref[...] + (pa_ref[...] + pb_ref[...])[None, :, :]
        u = jnp.mean(x, axis=-1, keepdims=True)
        s2 = jnp.mean((x - u) ** 2, axis=-1, keepdims=True)
        xn = (x - u) * lax.rsqrt(s2 + _EPS)
        o_ref[...] = xn * gm_ref[0][None, None, :] + bt_ref[0][None, None, :]

    return pl.pallas_call(
        body,
        grid=grid,
        in_specs=[
            pl.BlockSpec((RB, L, H), lambda i: (i, 0, 0)),
            pl.BlockSpec((L, H), lambda i: (0, 0)),
            pl.BlockSpec((L, H), lambda i: (0, 0)),
            pl.BlockSpec((1, H), lambda i: (0, 0)),
            pl.BlockSpec((1, H), lambda i: (0, 0)),
        ],
        out_specs=pl.BlockSpec((RB, L, H), lambda i: (i, 0, 0)),
        out_shape=jax.ShapeDtypeStruct((B, L, H), jnp.float32),
    )(gathered, pre_tab, pos_tab, gamma, beta)


def kernel(input_ids, word_table, pretrained_table, pos_table, gamma, beta):
    B, L = input_ids.shape
    H = word_table.shape[1]
    ids_flat = input_ids.reshape(-1).astype(jnp.int32)
    gathered = _sc_gather(ids_flat, word_table)
    return _tc_add_layernorm(
        gathered.reshape(B, L, H),
        pretrained_table[:L],
        pos_table[:L],
        gamma.reshape(1, H),
        beta.reshape(1, H),
    )
